# baseline (speedup 1.0000x reference)
import functools

import jax
import jax.numpy as jnp
from jax.experimental import pallas as pl
from jax.experimental.pallas import tpu as pltpu
from jax.experimental.pallas import tpu_sc as plsc

MAX_LEN = 512


def kernel(q, k, rel_pos_enc):
    seq_len_q = q.shape[1]
    seq_len_k = k.shape[1]
    d = rel_pos_enc.shape[1]

    info = plsc.get_sparse_core_info()
    nc, ns = info.num_cores, info.num_subcores
    nw = nc * ns
    rows_per_w = seq_len_q // nw

    mesh = plsc.VectorSubcoreMesh(core_axis_name="c", subcore_axis_name="s")

    @functools.partial(
        pl.kernel,
        mesh=mesh,
        out_type=jax.ShapeDtypeStruct((seq_len_q, seq_len_k, d), rel_pos_enc.dtype),
        compiler_params=pltpu.CompilerParams(use_tc_tiling_on_sc=False),
    )
    def run(table_hbm, out_hbm):
        wid = jax.lax.axis_index("s") * nc + jax.lax.axis_index("c")
        base = wid * rows_per_w

        def body(r, carry):
            i = base + r
            start = (MAX_LEN - 1) - i
            pltpu.sync_copy(table_hbm.at[pl.ds(start, seq_len_k), :],
                            out_hbm.at[i])
            return carry

        jax.lax.fori_loop(0, rows_per_w, body, 0)

    return run(rel_pos_enc)
